# baseline (speedup 1.0000x reference)
import functools

import jax
import jax.numpy as jnp
from jax import lax
from jax.experimental import pallas as pl
from jax.experimental.pallas import tpu as pltpu
from jax.experimental.pallas import tpu_sc as plsc

_D = 256
_NC = 8192
_NR = 16384
_BR = 512
_BC = 1024
_RB = _NR // _BR
_CB = _NC // _BC


def _argmin_body(x_ref, e_ref, eb_ref, ind_ref, diff_ref, s_ref, best_ref,
                 arg_ref):
    r = pl.program_id(0)
    c = pl.program_id(1)
    x = x_ref[...]
    e = e_ref[...]
    eb = eb_ref[...]

    @pl.when(c == 0)
    def _():
        s_ref[...] = jnp.sum(x * x, axis=1, keepdims=True)

    m = jnp.dot((2.0 * x).astype(jnp.bfloat16), eb,
                preferred_element_type=jnp.float32)
    se = jnp.sum(e * e, axis=0, keepdims=True)
    val = m - se

    lane = jax.lax.broadcasted_iota(jnp.int32, (_BR, 128), 1)

    @pl.when(c == 0)
    def _():
        best_ref[...] = jnp.full((_BR, 128), -jnp.inf, jnp.float32)
        arg_ref[...] = jnp.zeros((_BR, 128), jnp.int32)

    bv, bi = best_ref[...], arg_ref[...]
    for k in range(_BC // 128):
        cand = val[:, k * 128:(k + 1) * 128]
        cidx = lane + (c * _BC + k * 128)
        upd = cand > bv
        bv = jnp.where(upd, cand, bv)
        bi = jnp.where(upd, cidx, bi)
    best_ref[...] = bv
    arg_ref[...] = bi

    @pl.when(c == _CB - 1)
    def _():
        rowmax = jnp.max(bv, axis=1, keepdims=True)
        winner = jnp.min(jnp.where(bv == rowmax, bi, jnp.int32(2**30)),
                         axis=1).astype(jnp.int32)
        ind_ref[...] = winner

        @pl.when(r == 0)
        def _():
            diff_ref[...] = jnp.zeros_like(diff_ref)

        diff_ref[...] += jnp.sum(s_ref[...] - rowmax).reshape(1, 1)

    @pl.when((c == _CB - 1) & (r == _RB - 1))
    def _():
        diff_ref[...] = diff_ref[...] * (1.0 / (_NR * _D))


def _argmin_call(flatten, embed, embed_bf16, interpret=False):
    return pl.pallas_call(
        _argmin_body,
        grid=(_RB, _CB),
        in_specs=[
            pl.BlockSpec((_BR, _D), lambda r, c: (r, 0)),
            pl.BlockSpec((_D, _BC), lambda r, c: (0, c)),
            pl.BlockSpec((_D, _BC), lambda r, c: (0, c)),
        ],
        out_specs=[
            pl.BlockSpec((_BR,), lambda r, c: (r,)),
            pl.BlockSpec((1, 1), lambda r, c: (0, 0)),
        ],
        out_shape=[
            jax.ShapeDtypeStruct((_NR,), jnp.int32),
            jax.ShapeDtypeStruct((1, 1), jnp.float32),
        ],
        scratch_shapes=[
            pltpu.VMEM((_BR, 1), jnp.float32),
            pltpu.VMEM((_BR, 128), jnp.float32),
            pltpu.VMEM((_BR, 128), jnp.int32),
        ],
        interpret=interpret,
    )(flatten, embed, embed_bf16)


_NCORE = 2
_NSUB = 16
_NW = _NCORE * _NSUB
_BPW = _NR // _NW
_CH = 128
_NCH = _BPW // _CH


def _gather_body(table_hbm, idx_hbm, out_hbm, idx_v, rows_v, sem):
    wid = lax.axis_index("s") * _NCORE + lax.axis_index("c")
    base = wid * _BPW
    for j in range(_NCH):
        pltpu.sync_copy(idx_hbm.at[pl.ds(base + j * _CH, _CH)], idx_v)
        pltpu.async_copy(table_hbm.at[idx_v], rows_v, sem).wait()
        pltpu.sync_copy(rows_v, out_hbm.at[pl.ds(base + j * _CH, _CH)])


_gather = pl.kernel(
    _gather_body,
    out_type=jax.ShapeDtypeStruct((_NR, _D), jnp.float32),
    mesh=plsc.VectorSubcoreMesh(core_axis_name="c", subcore_axis_name="s"),
    scratch_types=[
        pltpu.VMEM((_CH,), jnp.int32),
        pltpu.VMEM((_CH, _D), jnp.float32),
        pltpu.SemaphoreType.DMA,
    ],
)


def kernel(input, embed):
    flatten = input.reshape(-1, _D)
    ind, diff = _argmin_call(flatten, embed, embed.astype(jnp.bfloat16))
    table = embed.T
    gathered = _gather(table, ind).reshape(input.shape)
    quantize = input + (gathered - input)
    return (
        quantize,
        diff[0, 0],
        ind.reshape(input.shape[:-1]),
    )


# speedup vs baseline: 1.0915x; 1.0915x over previous
import functools

import jax
import jax.numpy as jnp
from jax import lax
from jax.experimental import pallas as pl
from jax.experimental.pallas import tpu as pltpu
from jax.experimental.pallas import tpu_sc as plsc

_D = 256
_NC = 8192
_NR = 16384
_BR = 512
_BC = 1024
_RB = _NR // _BR
_CB = _NC // _BC


def _argmin_body(x_ref, e_ref, ind_ref, diff_ref, s_ref, best_ref, arg_ref):
    r = pl.program_id(0)
    c = pl.program_id(1)
    x = x_ref[...]
    e = e_ref[...]

    @pl.when(c == 0)
    def _():
        s_ref[...] = jnp.sum(x * x, axis=1, keepdims=True)

    m = jnp.dot((2.0 * x).astype(jnp.bfloat16), e.astype(jnp.bfloat16),
                preferred_element_type=jnp.float32)
    se = jnp.sum(e * e, axis=0, keepdims=True)
    val = m - se

    lane = jax.lax.broadcasted_iota(jnp.int32, (_BR, 128), 1)

    @pl.when(c == 0)
    def _():
        best_ref[...] = jnp.full((_BR, 128), -jnp.inf, jnp.float32)
        arg_ref[...] = jnp.zeros((_BR, 128), jnp.int32)

    bv, bi = best_ref[...], arg_ref[...]
    for k in range(_BC // 128):
        cand = val[:, k * 128:(k + 1) * 128]
        cidx = lane + (c * _BC + k * 128)
        upd = cand > bv
        bv = jnp.where(upd, cand, bv)
        bi = jnp.where(upd, cidx, bi)
    best_ref[...] = bv
    arg_ref[...] = bi

    @pl.when(c == _CB - 1)
    def _():
        rowmax = jnp.max(bv, axis=1, keepdims=True)
        winner = jnp.min(jnp.where(bv == rowmax, bi, jnp.int32(2**30)),
                         axis=1).astype(jnp.int32)
        ind_ref[...] = winner

        @pl.when(r == 0)
        def _():
            diff_ref[...] = jnp.zeros_like(diff_ref)

        diff_ref[...] += jnp.sum(s_ref[...] - rowmax).reshape(1, 1)

    @pl.when((c == _CB - 1) & (r == _RB - 1))
    def _():
        diff_ref[...] = diff_ref[...] * (1.0 / (_NR * _D))


def _argmin_call(flatten, embed, interpret=False):
    return pl.pallas_call(
        _argmin_body,
        grid=(_RB, _CB),
        in_specs=[
            pl.BlockSpec((_BR, _D), lambda r, c: (r, 0)),
            pl.BlockSpec((_D, _BC), lambda r, c: (0, c)),
        ],
        out_specs=[
            pl.BlockSpec((_BR,), lambda r, c: (r,)),
            pl.BlockSpec((1, 1), lambda r, c: (0, 0)),
        ],
        out_shape=[
            jax.ShapeDtypeStruct((_NR,), jnp.int32),
            jax.ShapeDtypeStruct((1, 1), jnp.float32),
        ],
        scratch_shapes=[
            pltpu.VMEM((_BR, 1), jnp.float32),
            pltpu.VMEM((_BR, 128), jnp.float32),
            pltpu.VMEM((_BR, 128), jnp.int32),
        ],
        interpret=interpret,
    )(flatten, embed)


_NCORE = 2
_NSUB = 16
_NW = _NCORE * _NSUB
_BPW = _NR // _NW
_CH = 128
_NCH = _BPW // _CH


def _gather_body(table_hbm, idx_hbm, out_hbm, idx_v, rows_v, sem):
    wid = lax.axis_index("s") * _NCORE + lax.axis_index("c")
    base = wid * _BPW
    for j in range(_NCH):
        pltpu.sync_copy(idx_hbm.at[pl.ds(base + j * _CH, _CH)], idx_v)
        pltpu.async_copy(table_hbm.at[idx_v], rows_v, sem).wait()
        pltpu.sync_copy(rows_v, out_hbm.at[pl.ds(base + j * _CH, _CH)])


_gather = pl.kernel(
    _gather_body,
    out_type=jax.ShapeDtypeStruct((_NR, _D), jnp.float32),
    mesh=plsc.VectorSubcoreMesh(core_axis_name="c", subcore_axis_name="s"),
    scratch_types=[
        pltpu.VMEM((_CH,), jnp.int32),
        pltpu.VMEM((_CH, _D), jnp.float32),
        pltpu.SemaphoreType.DMA,
    ],
)


def kernel(input, embed):
    flatten = input.reshape(-1, _D)
    ind, diff = _argmin_call(flatten, embed)
    table = embed.T
    gathered = _gather(table, ind).reshape(input.shape)
    quantize = input + (gathered - input)
    return (
        quantize,
        diff[0, 0],
        ind.reshape(input.shape[:-1]),
    )
